# baseline (device time: 72460 ns/iter reference)
import jax
import jax.numpy as jnp
from jax import lax
from jax.experimental import pallas as pl
from jax.experimental.pallas import tpu as pltpu

B, S, H, Dh, Dr = 2, 512, 16, 128, 32
D = 2048
DC_SH = 128
BS = B * S
SCALE = (Dh + Dr) ** -0.5
BF16 = jnp.bfloat16
NJ = 4
CHUNK = D // NJ
SCALE2 = SCALE * 1.4426950408889634


def _dot(a, b, out=jnp.float32):
    r = jnp.dot(a, b, preferred_element_type=jnp.float32)
    return r if out == jnp.float32 else r.astype(out)


def _dot_t(a, b):
    return lax.dot_general(a, b, (((1,), (1,)), ((), ())),
                           preferred_element_type=jnp.float32)


def _exchange_proj_body(
    x_ref, wdkv_ref, wuk_ref, wuv_ref, wq_ref, wqr_ref, wkr_ref,
    q_ref, qr_ref, kr_ref, c_loc, c_rem, wukl, wukr, wuvl, wuvr,
    send_sems, recv_sems,
):
    j = pl.program_id(0)
    my_x = lax.axis_index("x")
    my_y = lax.axis_index("y")
    my_z = lax.axis_index("z")
    peer = (1 - my_x, my_y, my_z)

    def mk(i, src, dst):
        return pltpu.make_async_remote_copy(
            src_ref=src, dst_ref=dst,
            send_sem=send_sems.at[i], recv_sem=recv_sems.at[i],
            device_id=peer, device_id_type=pl.DeviceIdType.MESH,
        )

    @pl.when(j == 0)
    def _():
        barrier = pltpu.get_barrier_semaphore()
        pl.semaphore_signal(barrier, inc=1, device_id=peer,
                            device_id_type=pl.DeviceIdType.MESH)
        pl.semaphore_wait(barrier, 1)
        wukl[...] = wuk_ref[...].astype(BF16)
        wuvl[...] = wuv_ref[...].astype(BF16)
        mk(1, wukl, wukr).start()
        mk(2, wuvl, wuvr).start()
        c_loc[...] = _dot(x_ref[...], wdkv_ref[...], out=BF16)
        mk(0, c_loc, c_rem).start()
        qr_ref[...] = (_dot(x_ref[...], wqr_ref[...]) * SCALE2).astype(BF16)
        kr_ref[...] = _dot(x_ref[...], wkr_ref[...], out=BF16)

    q_ref[...] = (_dot(x_ref[...], wq_ref[...]) * SCALE2).astype(BF16)

    @pl.when(j == NJ - 1)
    def _():
        for i, (src, dst) in enumerate(
            ((c_loc, c_rem), (wukl, wukr), (wuvl, wuvr))
        ):
            mk(i, src, dst).wait()


def _attn_out_body(q_ref, qr_ref, kr_ref, c_loc_ref, c_rem_ref,
                   wukl_ref, wukr_ref, wuvl_ref, wuvr_ref, wo_ref,
                   out_ref, o_scr, wo_bf):
    j = pl.program_id(0)
    b = pl.program_id(1)

    @pl.when(j == 0)
    def _():
        cl, cr = c_loc_ref[...], c_rem_ref[...]

        def _assemble(wl_ref, wr_ref):
            chunks = []
            for c0 in range(0, D, CHUNK):
                sl = slice(c0, c0 + CHUNK)
                chunks.append((_dot(cl, wl_ref[:, sl])
                               + _dot(cr, wr_ref[:, sl])).astype(BF16))
            return jnp.concatenate(chunks, axis=1)

        k = _assemble(wukl_ref, wukr_ref)
        v = _assemble(wuvl_ref, wuvr_ref)
        kr = kr_ref[...]
        o_cols = []
        for h in range(H):
            qh = q_ref[:, h * Dh:(h + 1) * Dh]
            qrh = qr_ref[:, h * Dr:(h + 1) * Dr]
            s = _dot_t(qh, k[:, h * Dh:(h + 1) * Dh]) + _dot_t(qrh, kr)
            p = jnp.exp2(s)
            recip = 1.0 / jnp.sum(p, axis=-1, keepdims=True)
            oh = _dot(p.astype(BF16), v[:, h * Dh:(h + 1) * Dh]) * recip
            o_cols.append(oh.astype(BF16))
        o_scr[pl.ds(b * S, S), :] = jnp.concatenate(o_cols, axis=1)

    @pl.when(b == 0)
    def _():
        wo_bf[...] = wo_ref[...].astype(BF16)

    out_ref[...] = _dot(o_scr[pl.ds(b * S, S), :], wo_bf[...])


def kernel(x, Wdkv, Wuk, Wuv, Wq, Wqr, Wkr, Wo):
    x2 = x.reshape(BS, D)

    q, qr, kr, c_loc, c_rem, wukl, wukr, wuvl, wuvr = pl.pallas_call(
        _exchange_proj_body,
        grid=(NJ,),
        out_shape=(
            jax.ShapeDtypeStruct((BS, D), BF16),
            jax.ShapeDtypeStruct((BS, H * Dr), BF16),
            jax.ShapeDtypeStruct((BS, Dr), BF16),
            jax.ShapeDtypeStruct((BS, DC_SH), BF16),
            jax.ShapeDtypeStruct((BS, DC_SH), BF16),
            jax.ShapeDtypeStruct((DC_SH, D), BF16),
            jax.ShapeDtypeStruct((DC_SH, D), BF16),
            jax.ShapeDtypeStruct((DC_SH, D), BF16),
            jax.ShapeDtypeStruct((DC_SH, D), BF16),
        ),
        in_specs=[
            pl.BlockSpec((BS, D), lambda j: (0, 0)),
            pl.BlockSpec((D, DC_SH), lambda j: (0, 0)),
            pl.BlockSpec((DC_SH, D), lambda j: (0, 0)),
            pl.BlockSpec((DC_SH, D), lambda j: (0, 0)),
            pl.BlockSpec((D, CHUNK), lambda j: (0, j)),
            pl.BlockSpec((D, H * Dr), lambda j: (0, 0)),
            pl.BlockSpec((D, Dr), lambda j: (0, 0)),
        ],
        out_specs=(
            pl.BlockSpec((BS, CHUNK), lambda j: (0, j)),
            pl.BlockSpec((BS, H * Dr), lambda j: (0, 0)),
            pl.BlockSpec((BS, Dr), lambda j: (0, 0)),
            pl.BlockSpec((BS, DC_SH), lambda j: (0, 0)),
            pl.BlockSpec((BS, DC_SH), lambda j: (0, 0)),
            pl.BlockSpec((DC_SH, D), lambda j: (0, 0)),
            pl.BlockSpec((DC_SH, D), lambda j: (0, 0)),
            pl.BlockSpec((DC_SH, D), lambda j: (0, 0)),
            pl.BlockSpec((DC_SH, D), lambda j: (0, 0)),
        ),
        scratch_shapes=[
            pltpu.SemaphoreType.DMA((3,)),
            pltpu.SemaphoreType.DMA((3,)),
        ],
        compiler_params=pltpu.CompilerParams(collective_id=0),
    )(x2, Wdkv, Wuk, Wuv, Wq, Wqr, Wkr)

    only_j0 = lambda j, b: (jnp.where(j == 0, b, 1), 0)
    out = pl.pallas_call(
        _attn_out_body,
        grid=(NJ, B),
        out_shape=jax.ShapeDtypeStruct((BS, D), jnp.float32),
        in_specs=[
            pl.BlockSpec((S, D), only_j0),
            pl.BlockSpec((S, H * Dr), only_j0),
            pl.BlockSpec((S, Dr), only_j0),
            pl.BlockSpec((S, DC_SH), only_j0),
            pl.BlockSpec((S, DC_SH), only_j0),
            pl.BlockSpec((DC_SH, D), lambda j, b: (0, 0)),
            pl.BlockSpec((DC_SH, D), lambda j, b: (0, 0)),
            pl.BlockSpec((DC_SH, D), lambda j, b: (0, 0)),
            pl.BlockSpec((DC_SH, D), lambda j, b: (0, 0)),
            pl.BlockSpec((D, CHUNK), lambda j, b: (0, j)),
        ],
        out_specs=pl.BlockSpec((S, CHUNK), lambda j, b: (b, j)),
        scratch_shapes=[
            pltpu.VMEM((BS, D), BF16),
            pltpu.VMEM((D, CHUNK), BF16),
        ],
        compiler_params=pltpu.CompilerParams(
            vmem_limit_bytes=48 * 1024 * 1024,
        ),
    )(q, qr, kr, c_loc, c_rem, wukl, wukr, wuvl, wuvr, Wo)

    return out.reshape(B, S, D)


# device time: 69452 ns/iter; 1.0433x vs baseline; 1.0433x over previous
import jax
import jax.numpy as jnp
from jax import lax
from jax.experimental import pallas as pl
from jax.experimental.pallas import tpu as pltpu

B, S, H, Dh, Dr = 2, 512, 16, 128, 32
D = 2048
DC_SH = 128
BS = B * S
SCALE = (Dh + Dr) ** -0.5
BF16 = jnp.bfloat16
NJ = 4
CHUNK = D // NJ
SCALE2 = SCALE * 1.4426950408889634


def _dot(a, b, out=jnp.float32):
    r = jnp.dot(a, b, preferred_element_type=jnp.float32)
    return r if out == jnp.float32 else r.astype(out)


def _dot_t(a, b):
    return lax.dot_general(a, b, (((1,), (1,)), ((), ())),
                           preferred_element_type=jnp.float32)


def _exchange_proj_body(
    x_ref, wdkv_ref, wuk_ref, wuv_ref, wq_ref, wqr_ref, wkr_ref,
    q_ref, qr_ref, kr_ref, c_loc, c_rem, wukl, wukr, wuvl, wuvr,
    send_sems, recv_sems,
):
    j = pl.program_id(0)
    my_x = lax.axis_index("x")
    my_y = lax.axis_index("y")
    my_z = lax.axis_index("z")
    peer = (1 - my_x, my_y, my_z)

    def mk(i, src, dst):
        return pltpu.make_async_remote_copy(
            src_ref=src, dst_ref=dst,
            send_sem=send_sems.at[i], recv_sem=recv_sems.at[i],
            device_id=peer, device_id_type=pl.DeviceIdType.MESH,
        )

    @pl.when(j == 0)
    def _():
        barrier = pltpu.get_barrier_semaphore()
        pl.semaphore_signal(barrier, inc=1, device_id=peer,
                            device_id_type=pl.DeviceIdType.MESH)
        pl.semaphore_wait(barrier, 1)
        wukl[...] = wuk_ref[...].astype(BF16)
        wuvl[...] = wuv_ref[...].astype(BF16)
        mk(1, wukl, wukr).start()
        mk(2, wuvl, wuvr).start()
        c_loc[...] = _dot(x_ref[...], wdkv_ref[...], out=BF16)
        mk(0, c_loc, c_rem).start()
        qr_ref[...] = (_dot(x_ref[...], wqr_ref[...]) * SCALE2).astype(BF16)
        kr_ref[...] = _dot(x_ref[...], wkr_ref[...], out=BF16)

    q_ref[...] = (_dot(x_ref[...], wq_ref[...]) * SCALE2).astype(BF16)

    @pl.when(j == NJ - 1)
    def _():
        for i, (src, dst) in enumerate(
            ((c_loc, c_rem), (wukl, wukr), (wuvl, wuvr))
        ):
            mk(i, src, dst).wait()


def _attn_body(q_ref, qr_ref, kr_ref, c_loc_ref, c_rem_ref,
               wukl_ref, wukr_ref, wuvl_ref, wuvr_ref, o_ref):
    cl, cr = c_loc_ref[...], c_rem_ref[...]

    def _assemble(wl_ref, wr_ref):
        chunks = []
        for c0 in range(0, D, CHUNK):
            sl = slice(c0, c0 + CHUNK)
            chunks.append((_dot(cl, wl_ref[:, sl])
                           + _dot(cr, wr_ref[:, sl])).astype(BF16))
        return jnp.concatenate(chunks, axis=1)

    k = _assemble(wukl_ref, wukr_ref)
    v = _assemble(wuvl_ref, wuvr_ref)
    kr = kr_ref[...]
    o_cols = []
    for h in range(H):
        qh = q_ref[:, h * Dh:(h + 1) * Dh]
        qrh = qr_ref[:, h * Dr:(h + 1) * Dr]
        s = _dot_t(qh, k[:, h * Dh:(h + 1) * Dh]) + _dot_t(qrh, kr)
        p = jnp.exp2(s)
        recip = 1.0 / jnp.sum(p, axis=-1, keepdims=True)
        oh = _dot(p.astype(BF16), v[:, h * Dh:(h + 1) * Dh]) * recip
        o_cols.append(oh.astype(BF16))
    o_ref[...] = jnp.concatenate(o_cols, axis=1)


def _out_body(o_ref, wo_ref, out_ref):
    out_ref[...] = _dot(o_ref[...], wo_ref[...].astype(BF16))


def kernel(x, Wdkv, Wuk, Wuv, Wq, Wqr, Wkr, Wo):
    x2 = x.reshape(BS, D)

    q, qr, kr, c_loc, c_rem, wukl, wukr, wuvl, wuvr = pl.pallas_call(
        _exchange_proj_body,
        grid=(NJ,),
        out_shape=(
            jax.ShapeDtypeStruct((BS, D), BF16),
            jax.ShapeDtypeStruct((BS, H * Dr), BF16),
            jax.ShapeDtypeStruct((BS, Dr), BF16),
            jax.ShapeDtypeStruct((BS, DC_SH), BF16),
            jax.ShapeDtypeStruct((BS, DC_SH), BF16),
            jax.ShapeDtypeStruct((DC_SH, D), BF16),
            jax.ShapeDtypeStruct((DC_SH, D), BF16),
            jax.ShapeDtypeStruct((DC_SH, D), BF16),
            jax.ShapeDtypeStruct((DC_SH, D), BF16),
        ),
        in_specs=[
            pl.BlockSpec((BS, D), lambda j: (0, 0)),
            pl.BlockSpec((D, DC_SH), lambda j: (0, 0)),
            pl.BlockSpec((DC_SH, D), lambda j: (0, 0)),
            pl.BlockSpec((DC_SH, D), lambda j: (0, 0)),
            pl.BlockSpec((D, CHUNK), lambda j: (0, j)),
            pl.BlockSpec((D, H * Dr), lambda j: (0, 0)),
            pl.BlockSpec((D, Dr), lambda j: (0, 0)),
        ],
        out_specs=(
            pl.BlockSpec((BS, CHUNK), lambda j: (0, j)),
            pl.BlockSpec((BS, H * Dr), lambda j: (0, 0)),
            pl.BlockSpec((BS, Dr), lambda j: (0, 0)),
            pl.BlockSpec((BS, DC_SH), lambda j: (0, 0)),
            pl.BlockSpec((BS, DC_SH), lambda j: (0, 0)),
            pl.BlockSpec((DC_SH, D), lambda j: (0, 0)),
            pl.BlockSpec((DC_SH, D), lambda j: (0, 0)),
            pl.BlockSpec((DC_SH, D), lambda j: (0, 0)),
            pl.BlockSpec((DC_SH, D), lambda j: (0, 0)),
        ),
        scratch_shapes=[
            pltpu.SemaphoreType.DMA((3,)),
            pltpu.SemaphoreType.DMA((3,)),
        ],
        compiler_params=pltpu.CompilerParams(collective_id=0),
    )(x2, Wdkv, Wuk, Wuv, Wq, Wqr, Wkr)

    o = pl.pallas_call(
        _attn_body,
        grid=(B,),
        out_shape=jax.ShapeDtypeStruct((BS, D), BF16),
        in_specs=[
            pl.BlockSpec((S, D), lambda b: (b, 0)),
            pl.BlockSpec((S, H * Dr), lambda b: (b, 0)),
            pl.BlockSpec((S, Dr), lambda b: (b, 0)),
            pl.BlockSpec((S, DC_SH), lambda b: (b, 0)),
            pl.BlockSpec((S, DC_SH), lambda b: (b, 0)),
            pl.BlockSpec((DC_SH, D), lambda b: (0, 0)),
            pl.BlockSpec((DC_SH, D), lambda b: (0, 0)),
            pl.BlockSpec((DC_SH, D), lambda b: (0, 0)),
            pl.BlockSpec((DC_SH, D), lambda b: (0, 0)),
        ],
        out_specs=pl.BlockSpec((S, D), lambda b: (b, 0)),
    )(q, qr, kr, c_loc, c_rem, wukl, wukr, wuvl, wuvr)

    out = pl.pallas_call(
        _out_body,
        grid=(NJ,),
        out_shape=jax.ShapeDtypeStruct((BS, D), jnp.float32),
        in_specs=[
            pl.BlockSpec((BS, D), lambda j: (0, 0)),
            pl.BlockSpec((D, CHUNK), lambda j: (0, j)),
        ],
        out_specs=pl.BlockSpec((BS, CHUNK), lambda j: (0, j)),
    )(o, Wo)

    return out.reshape(B, S, D)
